# baseline (speedup 1.0000x reference)
import numpy as np
import jax
import jax.numpy as jnp
from jax.experimental import pallas as pl
from jax.experimental.pallas import tpu as pltpu

_C_IN = 3
_C1 = 4
_C2 = 4
_K = 5
_H = 32
_H1 = 28
_P1 = 14
_H2 = 10
_P2 = 5
_F = _C2 * _P2 * _P2
_N_B = 512
_G = 4
_NG = _H1 // _G
_N1 = _G * _C1 * _H1 // 1

_COLS1 = 2 * _G * _C1 * _P1
_COLS2 = 2 * _H2 * _C2 * _P2
_HP1 = _COLS1 // 2
_HP2 = _COLS2 // 2
_W1_ROWS = (_G + _K - 1) * _H
_W2_ROWS = _P1 * _C2 * _P1


def _one_hot_factors():
    f32 = np.float32
    dy = np.arange(_K)
    a1 = (np.arange(_G + _K - 1)[None, :, None]
          == np.arange(_G)[None, None, :] + dy[:, None, None]).astype(f32)
    u = np.arange(_H)[None, :, None, None]
    b1f = (u == 2 * np.arange(_P1)[None, None, None, :]
           + np.arange(2)[None, None, :, None] + dy[:, None, None, None]
           ).astype(f32)
    a2 = (np.arange(_P1)[None, :, None]
          == np.arange(_H2)[None, None, :] + dy[:, None, None]).astype(f32)
    u2 = np.arange(_P1)[None, :, None, None]
    b2f = (u2 == 2 * np.arange(_P2)[None, None, None, :]
           + np.arange(2)[None, None, :, None] + dy[:, None, None, None]
           ).astype(f32)
    return a1, b1f, a2, b2f


_A1, _B1F, _A2, _B2F = _one_hot_factors()

_O_OF_COL1 = (np.arange(_COLS1) % (_C1 * _P1)) // _P1
_O_OF_COL2 = (np.arange(_COLS2) % (_C2 * _P2)) // _P2
_OB1 = (np.arange(_C1)[:, None] == _O_OF_COL1[None, :]).astype(np.float32)
_OB2 = (np.arange(_C2)[:, None] == _O_OF_COL2[None, :]).astype(np.float32)

_yp, _o, _xp = np.meshgrid(np.arange(_P2), np.arange(_C2), np.arange(_P2),
                           indexing="ij")
_FC_PERM = (_o * _P2 * _P2 + _yp * _P2 + _xp).ravel()
_PM = (np.arange(_F)[:, None] == _FC_PERM[None, :]).astype(np.float32)


def _encoder_body(x_ref, w1_ref, b1_ref, w2_ref, b2_ref,
                  wfc_ref, bfc_ref, wh_ref, bh_ref, out_ref, p1_ref):
    f32 = jnp.float32
    b1 = b1_ref[...]

    for g in range(_NG):
        h = b1
        for c in range(_C_IN):
            base = c * (_H * _H) + g * _G * _H
            h = h + jnp.dot(x_ref[:, base:base + _W1_ROWS], w1_ref[c],
                            preferred_element_type=f32)
        h = jnp.maximum(h, 0.0)
        v = jnp.maximum(h[:, :_HP1], h[:, _HP1:])
        q = _C1 * _P1
        p1_ref[:, (2 * g) * q:(2 * g + 1) * q] = (
            jnp.maximum(v[:, 0 * q:1 * q], v[:, 1 * q:2 * q]))
        p1_ref[:, (2 * g + 1) * q:(2 * g + 2) * q] = (
            jnp.maximum(v[:, 2 * q:3 * q], v[:, 3 * q:4 * q]))

    h2 = jnp.dot(p1_ref[...], w2_ref[...], preferred_element_type=f32)
    h2 = jnp.maximum(h2 + b2_ref[...], 0.0)
    v2 = jnp.maximum(h2[:, :_HP2], h2[:, _HP2:])
    q2 = _C2 * _P2
    f = jnp.concatenate(
        [jnp.maximum(v2[:, (2 * k) * q2:(2 * k + 1) * q2],
                     v2[:, (2 * k + 1) * q2:(2 * k + 2) * q2])
         for k in range(_P2)], axis=1)

    hid = jnp.dot(f, wfc_ref[...], preferred_element_type=f32) + bfc_ref[...]
    hid = jnp.maximum(hid, 0.0)
    out_ref[...] = (jnp.dot(hid, wh_ref[...], preferred_element_type=f32)
                    + bh_ref[...])


def kernel(state, w1, b1, w2, b2, fcw, fcb, muw, mub, vaw, vab):
    f32 = jnp.float32
    in_shape = state.shape
    x = state.astype(f32).reshape(-1, _C_IN * _H * _H)
    B = x.shape[0]
    L = muw.shape[0]

    nt = pl.cdiv(B, _N_B)
    bp = nt * _N_B
    if bp != B:
        x = jnp.pad(x, ((0, bp - B), (0, 0)))

    w1g = jnp.einsum("ocde,djr,eupx->cjuprox", w1.astype(f32),
                     _A1, _B1F).reshape(_C_IN, _W1_ROWS, _COLS1)
    w2f = jnp.einsum("ocde,djr,eupx->jcuprox", w2.astype(f32),
                     _A2, _B2F).reshape(_W2_ROWS, _COLS2)
    b1r = (b1.astype(f32) @ _OB1)[None, :]
    b2r = (b2.astype(f32) @ _OB2)[None, :]

    wfct = (fcw.astype(f32) @ _PM).T
    bfcr = fcb.astype(f32)[None, :]
    wht = jnp.concatenate([muw, vaw], axis=0).astype(f32).T
    bhr = jnp.concatenate([mub, vab]).astype(f32)[None, :]

    full2 = lambda t: (0, 0)
    out = pl.pallas_call(
        _encoder_body,
        grid=(nt,),
        in_specs=[
            pl.BlockSpec((_N_B, _C_IN * _H * _H), lambda t: (t, 0)),
            pl.BlockSpec((_C_IN, _W1_ROWS, _COLS1), lambda t: (0, 0, 0)),
            pl.BlockSpec((1, _COLS1), full2),
            pl.BlockSpec((_W2_ROWS, _COLS2), full2),
            pl.BlockSpec((1, _COLS2), full2),
            pl.BlockSpec((_F, _F), full2),
            pl.BlockSpec((1, _F), full2),
            pl.BlockSpec((_F, 2 * L), full2),
            pl.BlockSpec((1, 2 * L), full2),
        ],
        out_specs=pl.BlockSpec((_N_B, 2 * L), lambda t: (t, 0)),
        out_shape=jax.ShapeDtypeStruct((bp, 2 * L), f32),
        scratch_shapes=[pltpu.VMEM((_N_B, _P1 * _C1 * _P1), f32)],
        compiler_params=pltpu.CompilerParams(
            dimension_semantics=("parallel",),
            vmem_limit_bytes=40 * 1024 * 1024),
    )(x, w1g, b1r, w2f, b2r, wfct, bfcr, wht, bhr)

    mu = out[:B, :L].reshape(*in_shape[:-3], L)
    log_var = out[:B, L:].reshape(*in_shape[:-3], L)
    return mu, log_var


# speedup vs baseline: 3.0293x; 3.0293x over previous
import numpy as np
import jax
import jax.numpy as jnp
from jax.experimental import pallas as pl
from jax.experimental.pallas import tpu as pltpu

_C_IN = 3
_C1 = 4
_C2 = 4
_K = 5
_H = 32
_H1 = 28
_P1 = 14
_H2 = 10
_P2 = 5
_F = _C2 * _P2 * _P2
_N_B = 512
_G = 4

_NG = _H1 // _G
_ROWS1 = 2 * _G * _C1 * _P1
_HR1 = _ROWS1 // 2
_Q1 = _C1 * _P1
_KS1 = (_G + _K - 1) * _H
_ROWS2 = 2 * _H2 * _C2 * _P2
_HR2 = _ROWS2 // 2
_Q2 = _C2 * _P2
_KS2 = _P1 * _C2 * _P1


def _factors():
    f32 = np.float32
    dy = np.arange(_K)
    a1 = (np.arange(_G + _K - 1)[None, :, None]
          == np.arange(_G)[None, None, :] + dy[:, None, None]).astype(f32)
    b1 = (np.arange(_H)[None, :, None, None]
          == 2 * np.arange(_P1)[None, None, None, :]
          + np.arange(2)[None, None, :, None]
          + dy[:, None, None, None]).astype(f32)
    f1 = np.einsum("djr,eupx->deprxju", a1, b1).reshape(
        _K * _K, 2 * _G * _P1, _KS1)
    a2 = (np.arange(_P1)[None, :, None]
          == np.arange(_H2)[None, None, :] + dy[:, None, None]).astype(f32)
    b2 = (np.arange(_P1)[None, :, None, None]
          == 2 * np.arange(_P2)[None, None, None, :]
          + np.arange(2)[None, None, :, None]
          + dy[:, None, None, None]).astype(f32)
    f2 = np.einsum("dyr,eupx->deprxyu", a2, b2).reshape(
        _K * _K, 2 * _H2 * _P2, _P1 * _P1)
    return f1, f2


_F1, _F2 = _factors()

_OB1 = (np.arange(_C1)[:, None]
        == ((np.arange(_ROWS1) % _Q1) // _P1)[None, :]).astype(np.float32)
_OB2 = (np.arange(_C2)[:, None]
        == ((np.arange(_ROWS2) % _Q2) // _P2)[None, :]).astype(np.float32)

_yp, _o, _xp = np.meshgrid(np.arange(_P2), np.arange(_C2), np.arange(_P2),
                           indexing="ij")
_FC_PERM = (_o * _P2 * _P2 + _yp * _P2 + _xp).ravel()
_PM = (np.arange(_F)[:, None] == _FC_PERM[None, :]).astype(np.float32)


def _encoder_body(x_ref, w1_ref, b1_ref, w2_ref, b2_ref,
                  wfc_ref, bfc_ref, wh_ref, bh_ref, out_ref, p1_ref):
    f32 = jnp.float32
    b1 = b1_ref[...]

    for g in range(_NG):
        h = b1
        for c in range(_C_IN):
            base = c * (_H * _H) + g * _G * _H
            h = h + jnp.dot(w1_ref[c], x_ref[base:base + _KS1, :],
                            preferred_element_type=f32)
        h = jnp.maximum(h, 0.0)
        v = jnp.maximum(h[:_HR1, :], h[_HR1:, :])
        pa = jnp.maximum(v[0 * _Q1:1 * _Q1], v[1 * _Q1:2 * _Q1])
        pb = jnp.maximum(v[2 * _Q1:3 * _Q1], v[3 * _Q1:4 * _Q1])
        for o in range(_C1):
            r0 = o * (_P1 * _P1) + (2 * g) * _P1
            p1_ref[r0:r0 + _P1, :] = pa[o * _P1:(o + 1) * _P1, :]
            p1_ref[r0 + _P1:r0 + 2 * _P1, :] = pb[o * _P1:(o + 1) * _P1, :]

    h2 = jnp.dot(w2_ref[...], p1_ref[...], preferred_element_type=f32)
    h2 = jnp.maximum(h2 + b2_ref[...], 0.0)
    v2 = jnp.maximum(h2[:_HR2, :], h2[_HR2:, :])
    f = jnp.concatenate(
        [jnp.maximum(v2[(2 * k) * _Q2:(2 * k + 1) * _Q2],
                     v2[(2 * k + 1) * _Q2:(2 * k + 2) * _Q2])
         for k in range(_P2)], axis=0)

    hid = jnp.dot(wfc_ref[...], f, preferred_element_type=f32) + bfc_ref[...]
    hid = jnp.maximum(hid, 0.0)
    out_ref[...] = (jnp.dot(wh_ref[...], hid, preferred_element_type=f32)
                    + bh_ref[...])


def kernel(state, w1, b1, w2, b2, fcw, fcb, muw, mub, vaw, vab):
    f32 = jnp.float32
    in_shape = state.shape
    xt = state.astype(f32).reshape(-1, _C_IN * _H * _H).T
    B = xt.shape[1]
    L = muw.shape[0]

    nt = pl.cdiv(B, _N_B)
    bp = nt * _N_B
    if bp != B:
        xt = jnp.pad(xt, ((0, 0), (0, bp - B)))

    w1m = (w1.astype(f32).reshape(_C1 * _C_IN, _K * _K)
           @ jnp.asarray(_F1.reshape(_K * _K, -1)))
    w1g = w1m.reshape(_C1, _C_IN, 2 * _G, _P1, _KS1).transpose(
        1, 2, 0, 3, 4).reshape(_C_IN, _ROWS1, _KS1)
    w2m = (w2.astype(f32).reshape(_C2 * _C2, _K * _K)
           @ jnp.asarray(_F2.reshape(_K * _K, -1)))
    w2f = w2m.reshape(_C2, _C2, 2 * _H2, _P2, _P1 * _P1).transpose(
        2, 0, 3, 1, 4).reshape(_ROWS2, _KS2)
    b1r = (b1.astype(f32) @ jnp.asarray(_OB1))[:, None]
    b2r = (b2.astype(f32) @ jnp.asarray(_OB2))[:, None]

    wfcp = fcw.astype(f32) @ jnp.asarray(_PM)
    bfcr = fcb.astype(f32)[:, None]
    wh = jnp.concatenate([muw, vaw], axis=0).astype(f32)
    bhr = jnp.concatenate([mub, vab]).astype(f32)[:, None]

    full2 = lambda t: (0, 0)
    out = pl.pallas_call(
        _encoder_body,
        grid=(nt,),
        in_specs=[
            pl.BlockSpec((_C_IN * _H * _H, _N_B), lambda t: (0, t)),
            pl.BlockSpec((_C_IN, _ROWS1, _KS1), lambda t: (0, 0, 0)),
            pl.BlockSpec((_ROWS1, 1), full2),
            pl.BlockSpec((_ROWS2, _KS2), full2),
            pl.BlockSpec((_ROWS2, 1), full2),
            pl.BlockSpec((_F, _F), full2),
            pl.BlockSpec((_F, 1), full2),
            pl.BlockSpec((2 * L, _F), full2),
            pl.BlockSpec((2 * L, 1), full2),
        ],
        out_specs=pl.BlockSpec((2 * L, _N_B), lambda t: (0, t)),
        out_shape=jax.ShapeDtypeStruct((2 * L, bp), f32),
        scratch_shapes=[pltpu.VMEM((_KS2, _N_B), f32)],
        compiler_params=pltpu.CompilerParams(
            dimension_semantics=("parallel",),
            vmem_limit_bytes=40 * 1024 * 1024),
    )(xt, w1g, b1r, w2f, b2r, wfcp, bfcr, wh, bhr)

    mu = out[:L, :B].T.reshape(*in_shape[:-3], L)
    log_var = out[L:, :B].T.reshape(*in_shape[:-3], L)
    return mu, log_var


# speedup vs baseline: 3.7066x; 1.2236x over previous
import numpy as np
import jax
import jax.numpy as jnp
from jax.experimental import pallas as pl
from jax.experimental.pallas import tpu as pltpu

_C_IN = 3
_C1 = 4
_C2 = 4
_K = 5
_H = 32
_H1 = 28
_P1 = 14
_H2 = 10
_P2 = 5
_F = _C2 * _P2 * _P2
_N_B = 512
_G = 4

_NG = _H1 // _G
_M1 = 2 * _G * _P1
_KS1 = (_G + _K - 1) * _H
_M2 = 2 * _H2 * _P2
_KS2 = _P1 * _P1


def _factors():
    f32 = np.float32
    dy = np.arange(_K)
    a1 = (np.arange(_G + _K - 1)[None, :, None]
          == np.arange(_G)[None, None, :] + dy[:, None, None]).astype(f32)
    b1 = (np.arange(_H)[None, :, None, None]
          == 2 * np.arange(_P1)[None, None, None, :]
          + np.arange(2)[None, None, :, None]
          + dy[:, None, None, None]).astype(f32)
    f1 = np.einsum("djr,eupx->deprxju", a1, b1).reshape(_K * _K, _M1, _KS1)
    a2 = (np.arange(_P1)[None, :, None]
          == np.arange(_H2)[None, None, :] + dy[:, None, None]).astype(f32)
    b2 = (np.arange(_P1)[None, :, None, None]
          == 2 * np.arange(_P2)[None, None, None, :]
          + np.arange(2)[None, None, :, None]
          + dy[:, None, None, None]).astype(f32)
    f2 = np.einsum("dyr,eupx->deprxyu", a2, b2).reshape(_K * _K, _M2, _KS2)
    return f1, f2


_F1, _F2 = _factors()


def _encoder_body(x_ref, w1_ref, b1_ref, w2_ref, b2_ref,
                  wfc_ref, bfc_ref, wh_ref, bh_ref, out_ref, p1_ref):
    f32 = jnp.float32

    for g in range(_NG):
        h = None
        for c in range(_C_IN):
            base = c * (_H * _H) + g * _G * _H
            d = jnp.dot(w1_ref[:, c].reshape(_C1 * _M1, _KS1),
                        x_ref[base:base + _KS1, :],
                        preferred_element_type=f32)
            h = d if h is None else h + d
        for o in range(_C1):
            ho = h[o * _M1:(o + 1) * _M1, :]
            vo = jnp.maximum(
                jnp.maximum(ho[:_M1 // 2, :], ho[_M1 // 2:, :]) + b1_ref[o],
                0.0)
            r0 = o * (_P1 * _P1) + 2 * g * _P1
            p1_ref[r0:r0 + _P1, :] = (
                jnp.maximum(vo[0 * _P1:1 * _P1], vo[1 * _P1:2 * _P1]))
            p1_ref[r0 + _P1:r0 + 2 * _P1, :] = (
                jnp.maximum(vo[2 * _P1:3 * _P1], vo[3 * _P1:4 * _P1]))

    fs = []
    for o in range(_C2):
        h2 = None
        for c in range(_C1):
            d = jnp.dot(w2_ref[o, c], p1_ref[c * _KS2:(c + 1) * _KS2, :],
                        preferred_element_type=f32)
            h2 = d if h2 is None else h2 + d
        vo = jnp.maximum(
            jnp.maximum(h2[:_M2 // 2, :], h2[_M2 // 2:, :]) + b2_ref[o],
            0.0)
        for k in range(_P2):
            fs.append(jnp.maximum(vo[(2 * k) * _P2:(2 * k + 1) * _P2],
                                  vo[(2 * k + 1) * _P2:(2 * k + 2) * _P2]))
    f = jnp.concatenate(fs, axis=0)

    hid = jnp.dot(wfc_ref[...], f, preferred_element_type=f32) + bfc_ref[...]
    hid = jnp.maximum(hid, 0.0)
    out_ref[...] = (jnp.dot(wh_ref[...], hid, preferred_element_type=f32)
                    + bh_ref[...])


def kernel(state, w1, b1, w2, b2, fcw, fcb, muw, mub, vaw, vab):
    f32 = jnp.float32
    in_shape = state.shape
    xt = state.astype(f32).reshape(-1, _C_IN * _H * _H).T
    B = xt.shape[1]
    L = muw.shape[0]

    nt = pl.cdiv(B, _N_B)
    bp = nt * _N_B
    if bp != B:
        xt = jnp.pad(xt, ((0, 0), (0, bp - B)))

    w1g = (w1.astype(f32).reshape(_C1 * _C_IN, _K * _K)
           @ jnp.asarray(_F1.reshape(_K * _K, -1))
           ).reshape(_C1, _C_IN, _M1, _KS1)
    w2g = (w2.astype(f32).reshape(_C2 * _C2, _K * _K)
           @ jnp.asarray(_F2.reshape(_K * _K, -1))
           ).reshape(_C2, _C2, _M2, _KS2)

    wh = jnp.concatenate([muw, vaw], axis=0).astype(f32)
    bhr = jnp.concatenate([mub, vab]).astype(f32)[:, None]
    bfcr = fcb.astype(f32)[:, None]

    smem = pl.BlockSpec(memory_space=pltpu.MemorySpace.SMEM)
    full2 = lambda t: (0, 0)
    out = pl.pallas_call(
        _encoder_body,
        grid=(nt,),
        in_specs=[
            pl.BlockSpec((_C_IN * _H * _H, _N_B), lambda t: (0, t)),
            pl.BlockSpec((_C1, _C_IN, _M1, _KS1), lambda t: (0, 0, 0, 0)),
            smem,
            pl.BlockSpec((_C2, _C1, _M2, _KS2), lambda t: (0, 0, 0, 0)),
            smem,
            pl.BlockSpec((_F, _F), full2),
            pl.BlockSpec((_F, 1), full2),
            pl.BlockSpec((2 * L, _F), full2),
            pl.BlockSpec((2 * L, 1), full2),
        ],
        out_specs=pl.BlockSpec((2 * L, _N_B), lambda t: (0, t)),
        out_shape=jax.ShapeDtypeStruct((2 * L, bp), f32),
        scratch_shapes=[pltpu.VMEM((_C1 * _KS2, _N_B), f32)],
        compiler_params=pltpu.CompilerParams(
            dimension_semantics=("parallel",),
            vmem_limit_bytes=40 * 1024 * 1024),
    )(xt, w1g, b1.astype(f32), w2g, b2.astype(f32),
      fcw.astype(f32), bfcr, wh, bhr)

    mu = out[:L, :B].T.reshape(*in_shape[:-3], L)
    log_var = out[L:, :B].T.reshape(*in_shape[:-3], L)
    return mu, log_var


# speedup vs baseline: 3.7552x; 1.0131x over previous
import numpy as np
import jax
import jax.numpy as jnp
from jax.experimental import pallas as pl
from jax.experimental.pallas import tpu as pltpu

_C_IN = 3
_C1 = 4
_C2 = 4
_K = 5
_H = 32
_H1 = 28
_P1 = 14
_H2 = 10
_P2 = 5
_F = _C2 * _P2 * _P2
_N_B = 1024
_G = 4

_NG = _H1 // _G
_M1 = 2 * _G * _P1
_KS1 = (_G + _K - 1) * _H
_M2 = 2 * _H2 * _P2
_KS2 = _P1 * _P1


def _factors():
    f32 = np.float32
    dy = np.arange(_K)
    a1 = (np.arange(_G + _K - 1)[None, :, None]
          == np.arange(_G)[None, None, :] + dy[:, None, None]).astype(f32)
    b1 = (np.arange(_H)[None, :, None, None]
          == 2 * np.arange(_P1)[None, None, None, :]
          + np.arange(2)[None, None, :, None]
          + dy[:, None, None, None]).astype(f32)
    f1 = np.einsum("djr,eupx->deprxju", a1, b1).reshape(_K * _K, _M1, _KS1)
    a2 = (np.arange(_P1)[None, :, None]
          == np.arange(_H2)[None, None, :] + dy[:, None, None]).astype(f32)
    b2 = (np.arange(_P1)[None, :, None, None]
          == 2 * np.arange(_P2)[None, None, None, :]
          + np.arange(2)[None, None, :, None]
          + dy[:, None, None, None]).astype(f32)
    f2 = np.einsum("dyr,eupx->deprxyu", a2, b2).reshape(_K * _K, _M2, _KS2)
    return f1, f2


_F1, _F2 = _factors()


def _encoder_body(x_ref, w1_ref, b1_ref, w2_ref, b2_ref,
                  wfc_ref, bfc_ref, wh_ref, bh_ref, out_ref, p1_ref):
    f32 = jnp.float32

    for g in range(_NG):
        h = None
        for c in range(_C_IN):
            base = c * (_H * _H) + g * _G * _H
            d = jnp.dot(w1_ref[:, c].reshape(_C1 * _M1, _KS1),
                        x_ref[base:base + _KS1, :],
                        preferred_element_type=f32)
            h = d if h is None else h + d
        for o in range(_C1):
            ho = h[o * _M1:(o + 1) * _M1, :]
            vo = jnp.maximum(
                jnp.maximum(ho[:_M1 // 2, :], ho[_M1 // 2:, :]) + b1_ref[o],
                0.0)
            r0 = o * (_P1 * _P1) + 2 * g * _P1
            p1_ref[r0:r0 + _P1, :] = (
                jnp.maximum(vo[0 * _P1:1 * _P1], vo[1 * _P1:2 * _P1]))
            p1_ref[r0 + _P1:r0 + 2 * _P1, :] = (
                jnp.maximum(vo[2 * _P1:3 * _P1], vo[3 * _P1:4 * _P1]))

    fs = []
    for o in range(_C2):
        h2 = None
        for c in range(_C1):
            d = jnp.dot(w2_ref[o, c], p1_ref[c * _KS2:(c + 1) * _KS2, :],
                        preferred_element_type=f32)
            h2 = d if h2 is None else h2 + d
        vo = jnp.maximum(
            jnp.maximum(h2[:_M2 // 2, :], h2[_M2 // 2:, :]) + b2_ref[o],
            0.0)
        for k in range(_P2):
            fs.append(jnp.maximum(vo[(2 * k) * _P2:(2 * k + 1) * _P2],
                                  vo[(2 * k + 1) * _P2:(2 * k + 2) * _P2]))
    f = jnp.concatenate(fs, axis=0)

    hid = jnp.dot(wfc_ref[...], f, preferred_element_type=f32) + bfc_ref[...]
    hid = jnp.maximum(hid, 0.0)
    out_ref[...] = (jnp.dot(wh_ref[...], hid, preferred_element_type=f32)
                    + bh_ref[...])


def kernel(state, w1, b1, w2, b2, fcw, fcb, muw, mub, vaw, vab):
    f32 = jnp.float32
    in_shape = state.shape
    xt = state.astype(f32).reshape(-1, _C_IN * _H * _H).T
    B = xt.shape[1]
    L = muw.shape[0]

    nt = pl.cdiv(B, _N_B)
    bp = nt * _N_B
    if bp != B:
        xt = jnp.pad(xt, ((0, 0), (0, bp - B)))

    w1g = (w1.astype(f32).reshape(_C1 * _C_IN, _K * _K)
           @ jnp.asarray(_F1.reshape(_K * _K, -1))
           ).reshape(_C1, _C_IN, _M1, _KS1)
    w2g = (w2.astype(f32).reshape(_C2 * _C2, _K * _K)
           @ jnp.asarray(_F2.reshape(_K * _K, -1))
           ).reshape(_C2, _C2, _M2, _KS2)

    wh = jnp.concatenate([muw, vaw], axis=0).astype(f32)
    bhr = jnp.concatenate([mub, vab]).astype(f32)[:, None]
    bfcr = fcb.astype(f32)[:, None]

    smem = pl.BlockSpec(memory_space=pltpu.MemorySpace.SMEM)
    full2 = lambda t: (0, 0)
    out = pl.pallas_call(
        _encoder_body,
        grid=(nt,),
        in_specs=[
            pl.BlockSpec((_C_IN * _H * _H, _N_B), lambda t: (0, t)),
            pl.BlockSpec((_C1, _C_IN, _M1, _KS1), lambda t: (0, 0, 0, 0)),
            smem,
            pl.BlockSpec((_C2, _C1, _M2, _KS2), lambda t: (0, 0, 0, 0)),
            smem,
            pl.BlockSpec((_F, _F), full2),
            pl.BlockSpec((_F, 1), full2),
            pl.BlockSpec((2 * L, _F), full2),
            pl.BlockSpec((2 * L, 1), full2),
        ],
        out_specs=pl.BlockSpec((2 * L, _N_B), lambda t: (0, t)),
        out_shape=jax.ShapeDtypeStruct((2 * L, bp), f32),
        scratch_shapes=[pltpu.VMEM((_C1 * _KS2, _N_B), f32)],
        compiler_params=pltpu.CompilerParams(
            dimension_semantics=("parallel",),
            vmem_limit_bytes=40 * 1024 * 1024),
    )(xt, w1g, b1.astype(f32), w2g, b2.astype(f32),
      fcw.astype(f32), bfcr, wh, bhr)

    mu = out[:L, :B].T.reshape(*in_shape[:-3], L)
    log_var = out[L:, :B].T.reshape(*in_shape[:-3], L)
    return mu, log_var
